# baseline (speedup 1.0000x reference)
import functools

import jax
import jax.numpy as jnp
from jax import lax
from jax.experimental import pallas as pl
from jax.experimental.pallas import tpu as pltpu
from jax.experimental.pallas import tpu_sc as plsc

_VOCAB = 1000000
_DIM = 64
_B = 16384
_NC = 2
_NS = 16
_L = 16
_NW = _NC * _NS
_BPW = _B // _NW
_CHUNK = 128
_NCHUNK = _BPW // _CHUNK


def _body(word_hbm, ctx_hbm, w_hbm, c_hbm, out_hbm,
          widx, cidx, wrows, crows, outv, sem):
    wid = lax.axis_index("s") * _NC + lax.axis_index("c")
    base = wid * _BPW

    pltpu.sync_copy(word_hbm.at[pl.ds(base, _BPW)], widx)
    pltpu.sync_copy(ctx_hbm.at[pl.ds(base, _BPW)], cidx)

    copies = []
    for ch in range(_NCHUNK):
        s = pl.ds(ch * _CHUNK, _CHUNK)
        copies.append(pltpu.async_copy(
            w_hbm.at[widx.at[s]], wrows.at[s], sem))
        copies.append(pltpu.async_copy(
            c_hbm.at[cidx.at[s]], crows.at[s], sem))
    for c in copies:
        c.wait()

    lane = lax.iota(jnp.int32, _L)

    def blk_body(i, carry):
        b0 = i * _L
        out = jnp.zeros((_L,), jnp.float32)
        for j in range(_L):
            b = b0 + j
            acc = wrows[b, pl.ds(0, _L)] * crows[b, pl.ds(0, _L)]
            for k in range(1, _DIM // _L):
                acc = acc + wrows[b, pl.ds(k * _L, _L)] * crows[b, pl.ds(k * _L, _L)]
            for sh in (8, 4, 2, 1):
                acc = acc + jax.lax.gather(
                    acc, ((lane + sh) & (_L - 1))[:, None],
                    jax.lax.GatherDimensionNumbers(
                        offset_dims=(), collapsed_slice_dims=(0,),
                        start_index_map=(0,)),
                    (1,), mode=jax.lax.GatherScatterMode.PROMISE_IN_BOUNDS)
            out = jnp.where(lane == j, acc, out)
        outv[pl.ds(b0, _L)] = out
        return carry
    lax.fori_loop(0, _BPW // _L, blk_body, 0)

    pltpu.sync_copy(outv, out_hbm.at[pl.ds(base, _BPW)])


_mesh = plsc.VectorSubcoreMesh(core_axis_name="c", subcore_axis_name="s")

_sc_call = functools.partial(
    pl.kernel,
    out_type=jax.ShapeDtypeStruct((_B,), jnp.float32),
    mesh=_mesh,
    scratch_types=[
        pltpu.VMEM((_BPW,), jnp.int32),
        pltpu.VMEM((_BPW,), jnp.int32),
        pltpu.VMEM((_BPW, _DIM), jnp.float32),
        pltpu.VMEM((_BPW, _DIM), jnp.float32),
        pltpu.VMEM((_BPW,), jnp.float32),
        pltpu.SemaphoreType.DMA,
    ],
    compiler_params=pltpu.CompilerParams(use_tc_tiling_on_sc=False),
)(_body)


@jax.jit
def kernel(word, context, W, C):
    word = word.astype(jnp.int32)
    context = context.astype(jnp.int32)
    return _sc_call(word, context, W, C)


# speedup vs baseline: 1.5502x; 1.5502x over previous
import functools

import jax
import jax.numpy as jnp
from jax import lax
from jax.experimental import pallas as pl
from jax.experimental.pallas import tpu as pltpu
from jax.experimental.pallas import tpu_sc as plsc

_VOCAB = 1000000
_DIM = 64
_B = 16384
_NC = 2
_NS = 16
_L = 16
_NW = _NC * _NS
_BPW = _B // _NW
_K = 16


def _rot(v, lane, sh):
    return lax.gather(
        v, ((lane + sh) & (_L - 1))[:, None],
        lax.GatherDimensionNumbers(
            offset_dims=(), collapsed_slice_dims=(0,), start_index_map=(0,)),
        (1,), mode=lax.GatherScatterMode.PROMISE_IN_BOUNDS)


def _body(word_hbm, ctx_hbm, w_hbm, c_hbm, out_hbm,
          idx_v, widx_s, cidx_s, wrows, crows, outv, wsem, csem):
    wid = lax.axis_index("s") * _NC + lax.axis_index("c")
    base = wid * _BPW

    pltpu.sync_copy(word_hbm.at[pl.ds(base, _BPW)], widx_s)
    pltpu.sync_copy(ctx_hbm.at[pl.ds(base, _BPW)], cidx_s)
    del idx_v

    lane = lax.iota(jnp.int32, _L)

    wvec = widx_s[pl.ds(0, _L)]
    cvec = cidx_s[pl.ds(0, _L)]
    for j in range(_K):
        pltpu.async_copy(w_hbm.at[pl.ds(wvec[j], 1)], wrows.at[pl.ds(j, 1)],
                         wsem)
        pltpu.async_copy(c_hbm.at[pl.ds(cvec[j], 1)], crows.at[pl.ds(j, 1)],
                         csem)

    nblk = _BPW // _K

    def blk_body(i, carry):
        b0 = i * _K
        slot0 = (i % 2) * _K
        nslot0 = ((i + 1) % 2) * _K
        pltpu.make_async_copy(
            w_hbm.at[pl.ds(0, _K)], wrows.at[pl.ds(slot0, _K)], wsem).wait()
        pltpu.make_async_copy(
            c_hbm.at[pl.ds(0, _K)], crows.at[pl.ds(slot0, _K)], csem).wait()

        @pl.when(i + 1 < nblk)
        def _prefetch():
            wv = widx_s[pl.ds(b0 + _K, _L)]
            cv = cidx_s[pl.ds(b0 + _K, _L)]
            for j in range(_K):
                pltpu.async_copy(w_hbm.at[pl.ds(wv[j], 1)],
                                 wrows.at[pl.ds(nslot0 + j, 1)], wsem)
                pltpu.async_copy(c_hbm.at[pl.ds(cv[j], 1)],
                                 crows.at[pl.ds(nslot0 + j, 1)], csem)

        out = jnp.zeros((_L,), jnp.float32)
        for j in range(_K):
            s = slot0 + j
            acc = wrows[s, pl.ds(0, _L)] * crows[s, pl.ds(0, _L)]
            for k in range(1, _DIM // _L):
                acc = acc + wrows[s, pl.ds(k * _L, _L)] * crows[s, pl.ds(k * _L, _L)]
            for sh in (8, 4, 2, 1):
                acc = acc + _rot(acc, lane, sh)
            out = jnp.where(lane == j, acc, out)
        outv[pl.ds(b0, _K)] = out
        return carry

    lax.fori_loop(0, nblk, blk_body, 0)

    pltpu.sync_copy(outv, out_hbm.at[pl.ds(base, _BPW)])


_mesh = plsc.VectorSubcoreMesh(core_axis_name="c", subcore_axis_name="s")

_sc_call = functools.partial(
    pl.kernel,
    out_type=jax.ShapeDtypeStruct((_B,), jnp.float32),
    mesh=_mesh,
    scratch_types=[
        pltpu.VMEM((_BPW,), jnp.int32),
        pltpu.VMEM((_BPW,), jnp.int32),
        pltpu.VMEM((_BPW,), jnp.int32),
        pltpu.VMEM((2 * _K, _DIM), jnp.float32),
        pltpu.VMEM((2 * _K, _DIM), jnp.float32),
        pltpu.VMEM((_BPW,), jnp.float32),
        pltpu.SemaphoreType.DMA,
        pltpu.SemaphoreType.DMA,
    ],
)(_body)


@jax.jit
def kernel(word, context, W, C):
    word = word.astype(jnp.int32)
    context = context.astype(jnp.int32)
    return _sc_call(word, context, W, C)
